# Initial kernel scaffold; baseline (speedup 1.0000x reference)
import jax
import jax.numpy as jnp
from jax.experimental import pallas as pl


def kernel(x, W, b):
    raise NotImplementedError("write your pallas kernel here")


# speedup vs baseline: 1.0421x; 1.0421x over previous
import jax
import jax.numpy as jnp
from jax.experimental import pallas as pl
from jax.experimental.pallas import tpu as pltpu

NUM_EXPERTS = 64
TOP_K = 8
HIDDEN = 4096
TOKENS = 32768

BT = 512


def _router_body(x_ref, wt_ref_in, b_ref, logits_ref, w_ref, idx_ref, mask_ref):
    x = x_ref[...]
    wt = wt_ref_in[...]
    logits = jnp.dot(x, wt, preferred_element_type=jnp.float32)
    logits = logits + b_ref[...]
    logits_ref[...] = logits

    iota = jax.lax.broadcasted_iota(jnp.int32, (BT, NUM_EXPERTS), 1)
    l = logits
    vals = []
    idxs = []
    for _ in range(TOP_K):
        m = jnp.max(l, axis=1, keepdims=True)
        eq = l == m
        idx = jnp.min(jnp.where(eq, iota, NUM_EXPERTS), axis=1,
                      keepdims=True)
        sel = iota == idx
        l = jnp.where(sel, -jnp.inf, l)
        vals.append(m)
        idxs.append(idx)
    v = jnp.concatenate(vals, axis=1)
    ii = jnp.concatenate(idxs, axis=1)

    e = jnp.exp(v - v[:, :1])
    w_ref[...] = e / jnp.sum(e, axis=1, keepdims=True)
    idx_ref[...] = ii

    ii_t = ii.T
    eiota = jax.lax.broadcasted_iota(jnp.int32, (NUM_EXPERTS, TOP_K, BT), 0)
    mask_ref[...] = (eiota == ii_t[None, :, :]).astype(jnp.int32)


def kernel(x, W, b):
    wt = W.T
    b2 = b.reshape(1, NUM_EXPERTS)
    grid = (TOKENS // BT,)
    out = pl.pallas_call(
        _router_body,
        grid=grid,
        in_specs=[
            pl.BlockSpec((BT, HIDDEN), lambda i: (i, 0)),
            pl.BlockSpec((HIDDEN, NUM_EXPERTS), lambda i: (0, 0)),
            pl.BlockSpec((1, NUM_EXPERTS), lambda i: (0, 0)),
        ],
        out_specs=[
            pl.BlockSpec((BT, NUM_EXPERTS), lambda i: (i, 0)),
            pl.BlockSpec((BT, TOP_K), lambda i: (i, 0)),
            pl.BlockSpec((BT, TOP_K), lambda i: (i, 0)),
            pl.BlockSpec((NUM_EXPERTS, TOP_K, BT), lambda i: (0, 0, i)),
        ],
        out_shape=[
            jax.ShapeDtypeStruct((TOKENS, NUM_EXPERTS), jnp.float32),
            jax.ShapeDtypeStruct((TOKENS, TOP_K), jnp.float32),
            jax.ShapeDtypeStruct((TOKENS, TOP_K), jnp.int32),
            jax.ShapeDtypeStruct((NUM_EXPERTS, TOP_K, TOKENS), jnp.int32),
        ],
        compiler_params=pltpu.CompilerParams(
            dimension_semantics=("arbitrary",),
        ),
    )(x, wt, b2)
    return (out[0], out[1], out[2], out[3])


# speedup vs baseline: 1.3707x; 1.3153x over previous
import jax
import jax.numpy as jnp
from jax.experimental import pallas as pl
from jax.experimental.pallas import tpu as pltpu

NUM_EXPERTS = 64
TOP_K = 8
HIDDEN = 4096
TOKENS = 32768

BT = 512
NB = TOKENS // BT


def _router_body(x_ref, wt_ref, b_ref, logits_ref, w_ref, idx_ref, mask_ref,
                 sc0_ref, sc1_ref):
    i = pl.program_id(0)

    prev_par = jax.lax.rem(i + 1, 2)
    l = jnp.where(prev_par == 0, sc0_ref[...], sc1_ref[...])
    eio = jax.lax.broadcasted_iota(jnp.int32, (NUM_EXPERTS, BT), 0)
    vals, idxs = [], []
    for r in range(TOP_K):
        m = jnp.max(l, axis=0, keepdims=True)
        eq = l == m
        idx = jnp.min(jnp.where(eq, eio, NUM_EXPERTS), axis=0,
                      keepdims=True)
        mask_ref[:, r, :] = eq.astype(jnp.int32)
        l = jnp.where(eq, -jnp.inf, l)
        vals.append(m)
        idxs.append(idx)
    v = jnp.concatenate(vals, axis=0)
    ii = jnp.concatenate(idxs, axis=0)
    e = jnp.exp(v - v[0:1])
    w = e / jnp.sum(e, axis=0, keepdims=True)
    w_ref[...] = w.T
    idx_ref[...] = ii.T

    logits = jnp.dot(x_ref[...], wt_ref[...],
                     preferred_element_type=jnp.float32) + b_ref[...]
    logits_ref[...] = logits
    lt = logits.T
    par = jax.lax.rem(i, 2)

    @pl.when(par == 0)
    def _():
        sc0_ref[...] = lt

    @pl.when(par == 1)
    def _():
        sc1_ref[...] = lt


def kernel(x, W, b):
    wt = W.T
    b2 = b.reshape(1, NUM_EXPERTS)
    out = pl.pallas_call(
        _router_body,
        grid=(NB + 1,),
        in_specs=[
            pl.BlockSpec((BT, HIDDEN), lambda i: (jnp.minimum(i, NB - 1), 0)),
            pl.BlockSpec((HIDDEN, NUM_EXPERTS), lambda i: (0, 0)),
            pl.BlockSpec((1, NUM_EXPERTS), lambda i: (0, 0)),
        ],
        out_specs=[
            pl.BlockSpec((BT, NUM_EXPERTS),
                         lambda i: (jnp.minimum(i, NB - 1), 0)),
            pl.BlockSpec((BT, TOP_K), lambda i: (jnp.maximum(i - 1, 0), 0)),
            pl.BlockSpec((BT, TOP_K), lambda i: (jnp.maximum(i - 1, 0), 0)),
            pl.BlockSpec((NUM_EXPERTS, TOP_K, BT),
                         lambda i: (0, 0, jnp.maximum(i - 1, 0))),
        ],
        out_shape=[
            jax.ShapeDtypeStruct((TOKENS, NUM_EXPERTS), jnp.float32),
            jax.ShapeDtypeStruct((TOKENS, TOP_K), jnp.float32),
            jax.ShapeDtypeStruct((TOKENS, TOP_K), jnp.int32),
            jax.ShapeDtypeStruct((NUM_EXPERTS, TOP_K, TOKENS), jnp.int32),
        ],
        scratch_shapes=[
            pltpu.VMEM((NUM_EXPERTS, BT), jnp.float32),
            pltpu.VMEM((NUM_EXPERTS, BT), jnp.float32),
        ],
        compiler_params=pltpu.CompilerParams(
            dimension_semantics=("arbitrary",),
        ),
    )(x, wt, b2)
    return (out[0], out[1], out[2], out[3])


# speedup vs baseline: 1.3976x; 1.0196x over previous
import jax
import jax.numpy as jnp
from jax.experimental import pallas as pl
from jax.experimental.pallas import tpu as pltpu

NUM_EXPERTS = 64
TOP_K = 8
HIDDEN = 4096
TOKENS = 32768

BT = 1024
NB = TOKENS // BT


def _router_body(x_ref, wt_ref, b_ref, logits_ref, w_ref, idx_ref, mask_ref,
                 sc0_ref, sc1_ref):
    i = pl.program_id(0)

    prev_par = jax.lax.rem(i + 1, 2)
    l = jnp.where(prev_par == 0, sc0_ref[...], sc1_ref[...])
    eio = jax.lax.broadcasted_iota(jnp.int32, (NUM_EXPERTS, BT), 0)
    vals, idxs = [], []
    for r in range(TOP_K):
        m = jnp.max(l, axis=0, keepdims=True)
        eq = l == m
        idx = jnp.min(jnp.where(eq, eio, NUM_EXPERTS), axis=0,
                      keepdims=True)
        mask_ref[:, r, :] = eq.astype(jnp.int32)
        l = jnp.where(eq, -jnp.inf, l)
        vals.append(m)
        idxs.append(idx)
    v = jnp.concatenate(vals, axis=0)
    ii = jnp.concatenate(idxs, axis=0)
    e = jnp.exp(v - v[0:1])
    w = e / jnp.sum(e, axis=0, keepdims=True)
    w_ref[...] = w.T
    idx_ref[...] = ii.T

    logits = jnp.dot(x_ref[...], wt_ref[...],
                     preferred_element_type=jnp.float32) + b_ref[...]
    logits_ref[...] = logits
    lt = logits.T
    par = jax.lax.rem(i, 2)

    @pl.when(par == 0)
    def _():
        sc0_ref[...] = lt

    @pl.when(par == 1)
    def _():
        sc1_ref[...] = lt


def kernel(x, W, b):
    wt = W.T
    b2 = b.reshape(1, NUM_EXPERTS)
    out = pl.pallas_call(
        _router_body,
        grid=(NB + 1,),
        in_specs=[
            pl.BlockSpec((BT, HIDDEN), lambda i: (jnp.minimum(i, NB - 1), 0)),
            pl.BlockSpec((HIDDEN, NUM_EXPERTS), lambda i: (0, 0)),
            pl.BlockSpec((1, NUM_EXPERTS), lambda i: (0, 0)),
        ],
        out_specs=[
            pl.BlockSpec((BT, NUM_EXPERTS),
                         lambda i: (jnp.minimum(i, NB - 1), 0)),
            pl.BlockSpec((BT, TOP_K), lambda i: (jnp.maximum(i - 1, 0), 0)),
            pl.BlockSpec((BT, TOP_K), lambda i: (jnp.maximum(i - 1, 0), 0)),
            pl.BlockSpec((NUM_EXPERTS, TOP_K, BT),
                         lambda i: (0, 0, jnp.maximum(i - 1, 0))),
        ],
        out_shape=[
            jax.ShapeDtypeStruct((TOKENS, NUM_EXPERTS), jnp.float32),
            jax.ShapeDtypeStruct((TOKENS, TOP_K), jnp.float32),
            jax.ShapeDtypeStruct((TOKENS, TOP_K), jnp.int32),
            jax.ShapeDtypeStruct((NUM_EXPERTS, TOP_K, TOKENS), jnp.int32),
        ],
        scratch_shapes=[
            pltpu.VMEM((NUM_EXPERTS, BT), jnp.float32),
            pltpu.VMEM((NUM_EXPERTS, BT), jnp.float32),
        ],
        compiler_params=pltpu.CompilerParams(
            dimension_semantics=("arbitrary",),
        ),
    )(x, wt, b2)
    return (out[0], out[1], out[2], out[3])


# speedup vs baseline: 1.4170x; 1.0138x over previous
import jax
import jax.numpy as jnp
from jax.experimental import pallas as pl
from jax.experimental.pallas import tpu as pltpu

NUM_EXPERTS = 64
TOP_K = 8
HIDDEN = 4096
TOKENS = 32768

BT = 1024
HBT = BT // 2
NB = TOKENS // BT


def _router_body(xa_ref, xb_ref, wt_ref, b_ref, logits_ref, w_ref, idx_ref,
                 mask_ref, sc0_ref, sc1_ref):
    i = pl.program_id(0)

    prev_par = jax.lax.rem(i + 1, 2)
    l = jnp.where(prev_par == 0, sc0_ref[...], sc1_ref[...])
    eio = jax.lax.broadcasted_iota(jnp.int32, (NUM_EXPERTS, BT), 0)
    vals, idxs = [], []
    for r in range(TOP_K):
        m = jnp.max(l, axis=0, keepdims=True)
        eq = l == m
        idx = jnp.min(jnp.where(eq, eio, NUM_EXPERTS), axis=0,
                      keepdims=True)
        mask_ref[:, r, :] = eq.astype(jnp.int32)
        l = jnp.where(eq, -jnp.inf, l)
        vals.append(m)
        idxs.append(idx)
    v = jnp.concatenate(vals, axis=0)
    ii = jnp.concatenate(idxs, axis=0)
    e = jnp.exp(v - v[0:1])
    w = e / jnp.sum(e, axis=0, keepdims=True)
    w_ref[...] = w.T
    idx_ref[...] = ii.T

    wt = wt_ref[...]
    la = jnp.dot(xa_ref[...], wt, preferred_element_type=jnp.float32) + b_ref[...]
    lb = jnp.dot(xb_ref[...], wt, preferred_element_type=jnp.float32) + b_ref[...]
    logits_ref[:HBT, :] = la
    logits_ref[HBT:, :] = lb
    par = jax.lax.rem(i, 2)

    @pl.when(par == 0)
    def _():
        sc0_ref[:, :HBT] = la.T
        sc0_ref[:, HBT:] = lb.T

    @pl.when(par == 1)
    def _():
        sc1_ref[:, :HBT] = la.T
        sc1_ref[:, HBT:] = lb.T


def kernel(x, W, b):
    wt = W.T
    b2 = b.reshape(1, NUM_EXPERTS)
    out = pl.pallas_call(
        _router_body,
        grid=(NB + 1,),
        in_specs=[
            pl.BlockSpec((HBT, HIDDEN),
                         lambda i: (2 * jnp.minimum(i, NB - 1), 0)),
            pl.BlockSpec((HBT, HIDDEN),
                         lambda i: (2 * jnp.minimum(i, NB - 1) + 1, 0)),
            pl.BlockSpec((HIDDEN, NUM_EXPERTS), lambda i: (0, 0)),
            pl.BlockSpec((1, NUM_EXPERTS), lambda i: (0, 0)),
        ],
        out_specs=[
            pl.BlockSpec((BT, NUM_EXPERTS),
                         lambda i: (jnp.minimum(i, NB - 1), 0)),
            pl.BlockSpec((BT, TOP_K), lambda i: (jnp.maximum(i - 1, 0), 0)),
            pl.BlockSpec((BT, TOP_K), lambda i: (jnp.maximum(i - 1, 0), 0)),
            pl.BlockSpec((NUM_EXPERTS, TOP_K, BT),
                         lambda i: (0, 0, jnp.maximum(i - 1, 0))),
        ],
        out_shape=[
            jax.ShapeDtypeStruct((TOKENS, NUM_EXPERTS), jnp.float32),
            jax.ShapeDtypeStruct((TOKENS, TOP_K), jnp.float32),
            jax.ShapeDtypeStruct((TOKENS, TOP_K), jnp.int32),
            jax.ShapeDtypeStruct((NUM_EXPERTS, TOP_K, TOKENS), jnp.int32),
        ],
        scratch_shapes=[
            pltpu.VMEM((NUM_EXPERTS, BT), jnp.float32),
            pltpu.VMEM((NUM_EXPERTS, BT), jnp.float32),
        ],
        compiler_params=pltpu.CompilerParams(
            dimension_semantics=("arbitrary",),
        ),
    )(x, x, wt, b2)
    return (out[0], out[1], out[2], out[3])
